# Initial kernel scaffold; baseline (speedup 1.0000x reference)
import functools

import jax
import jax.numpy as jnp
from jax import lax
from jax.experimental import pallas as pl
from jax.experimental.pallas import tpu as pltpu
from jax.experimental.pallas import tpu_sc as plsc

_R = 32
_R3 = _R * _R * _R
_B = 4
_C = 64
_N = 100000
_CHUNK = 2000
_NCHUNK = _N // _CHUNK
_NC = 2
_NS = 16
_NW = _NC * _NS
_WPB = _NW // _B
_CPW = _C // _WPB
_L = 16


def _prep_body(c_ref, nc_ref, idx_ref):
    c = c_ref[0]
    mean = jnp.mean(c, axis=1, keepdims=True)
    nrm = c - mean
    sumsq = jnp.sum(nrm * nrm, axis=0, keepdims=True)
    rmax = jnp.sqrt(jnp.max(sumsq))
    ncoords = nrm / (rmax * 2.0) + 0.5
    nc_ref[0] = ncoords
    scaled = jnp.clip(ncoords * _R, 0.0, _R - 1.0)
    vox = jnp.round(scaled).astype(jnp.int32)
    idx = vox[0:1] * (_R * _R) + vox[1:2] * _R + vox[2:3]
    idx_ref[0] = idx


def _prep(coords):
    return pl.pallas_call(
        _prep_body,
        grid=(_B,),
        in_specs=[pl.BlockSpec((1, 3, _N), lambda b: (b, 0, 0))],
        out_specs=[
            pl.BlockSpec((1, 3, _N), lambda b: (b, 0, 0)),
            pl.BlockSpec((1, 1, _N), lambda b: (b, 0, 0)),
        ],
        out_shape=[
            jax.ShapeDtypeStruct((_B, 3, _N), jnp.float32),
            jax.ShapeDtypeStruct((_B, 1, _N), jnp.int32),
        ],
    )(coords)


def _sc_body(feat_hbm, idx_hbm, out_hbm, acc, invc, idxb, featb):
    cid = lax.axis_index("c")
    sid = lax.axis_index("s")
    wid = sid * _NC + cid
    b = wid // _WPB
    g = wid % _WPB

    def _zero_acc():
        @pl.loop(0, _R3, step=_L)
        def _(i):
            acc[pl.ds(i, _L)] = jnp.zeros((_L,), jnp.float32)

    ones = jnp.ones((_L,), jnp.float32)

    _zero_acc()

    @pl.loop(0, _NCHUNK)
    def _(ch):
        pltpu.sync_copy(idx_hbm.at[b, pl.ds(ch * _CHUNK, _CHUNK)], idxb)

        @pl.loop(0, _CHUNK, step=_L)
        def _(i):
            vidx = idxb[pl.ds(i, _L)]
            plsc.addupdate_scatter(acc, [vidx], ones)

    @pl.loop(0, _R3, step=_L)
    def _(i):
        invc[pl.ds(i, _L)] = 1.0 / jnp.maximum(acc[pl.ds(i, _L)], 1.0)

    @pl.loop(0, _CPW)
    def _(k):
        ci = g * _CPW + k
        _zero_acc()

        @pl.loop(0, _NCHUNK)
        def _(ch):
            pltpu.sync_copy(idx_hbm.at[b, pl.ds(ch * _CHUNK, _CHUNK)], idxb)
            pltpu.sync_copy(feat_hbm.at[b, ci, pl.ds(ch * _CHUNK, _CHUNK)], featb)

            @pl.loop(0, _CHUNK, step=_L)
            def _(i):
                vidx = idxb[pl.ds(i, _L)]
                v = featb[pl.ds(i, _L)]
                plsc.addupdate_scatter(acc, [vidx], v)

        @pl.loop(0, _R3, step=_L)
        def _(i):
            acc[pl.ds(i, _L)] = acc[pl.ds(i, _L)] * invc[pl.ds(i, _L)]

        pltpu.sync_copy(acc, out_hbm.at[b, ci])


_sc_seg = pl.kernel(
    _sc_body,
    out_type=jax.ShapeDtypeStruct((_B, _C, _R3), jnp.float32),
    mesh=plsc.VectorSubcoreMesh(core_axis_name="c", subcore_axis_name="s"),
    scratch_types=[
        pltpu.VMEM((_R3,), jnp.float32),
        pltpu.VMEM((_R3,), jnp.float32),
        pltpu.VMEM((_CHUNK,), jnp.int32),
        pltpu.VMEM((_CHUNK,), jnp.float32),
    ],
)


def kernel(features, coords):
    ncoords, idx = _prep(coords)
    segavg = _sc_seg(features, idx.reshape(_B, _N))
    return segavg.reshape(_B, _C, _R, _R, _R), ncoords


# speedup vs baseline: 1.2229x; 1.2229x over previous
import dataclasses
import functools

import jax
import jax.numpy as jnp
from jax import lax
from jax.experimental import pallas as pl
from jax.experimental.pallas import tpu as pltpu
from jax.experimental.pallas import tpu_sc as plsc

_R = 32
_R3 = _R * _R * _R
_B = 4
_C = 64
_N = 100000
_CHUNK = 2000
_NCHUNK = _N // _CHUNK
_NC = 2
_NS = 16
_NW = _NC * _NS
_WPB = _NW // _B
_CPW = _C // _WPB
_L = 16


def _prep_body(c_ref, nc_ref, idx_ref):
    c = c_ref[0]
    mean = jnp.mean(c, axis=1, keepdims=True)
    nrm = c - mean
    sumsq = jnp.sum(nrm * nrm, axis=0, keepdims=True)
    rmax = jnp.sqrt(jnp.max(sumsq))
    ncoords = nrm / (rmax * 2.0) + 0.5
    scaled = jnp.clip(ncoords * _R, 0.0, _R - 1.0)
    nc_ref[0] = scaled
    vox = jnp.round(scaled).astype(jnp.int32)
    idx = vox[0:1] * (_R * _R) + vox[1:2] * _R + vox[2:3]
    idx_ref[0] = idx


def _prep(coords):
    return pl.pallas_call(
        _prep_body,
        grid=(_B,),
        in_specs=[pl.BlockSpec((1, 3, _N), lambda b: (b, 0, 0))],
        out_specs=[
            pl.BlockSpec((1, 3, _N), lambda b: (b, 0, 0)),
            pl.BlockSpec((1, 1, _N), lambda b: (b, 0, 0)),
        ],
        out_shape=[
            jax.ShapeDtypeStruct((_B, 3, _N), jnp.float32),
            jax.ShapeDtypeStruct((_B, 1, _N), jnp.int32),
        ],
    )(coords)


def _sc_body(feat_hbm, idx_hbm, out_hbm, acc, invc, idxb, featb):
    cid = lax.axis_index("c")
    sid = lax.axis_index("s")
    wid = sid * _NC + cid
    b = wid // _WPB
    g = wid % _WPB

    def _zero_acc():
        @pl.loop(0, _R3, step=_L)
        def _(i):
            acc[pl.ds(i, _L)] = jnp.zeros((_L,), jnp.float32)

    ones = jnp.ones((_L,), jnp.float32)

    _zero_acc()

    @pl.loop(0, _NCHUNK)
    def _(ch):
        pltpu.sync_copy(idx_hbm.at[pl.ds(b * _N + ch * _CHUNK, _CHUNK)], idxb)

        @pl.loop(0, _CHUNK, step=_L)
        def _(i):
            vidx = idxb[pl.ds(i, _L)]
            plsc.addupdate_scatter(acc, [vidx], ones)

    @pl.loop(0, _R3, step=_L)
    def _(i):
        invc[pl.ds(i, _L)] = 1.0 / jnp.maximum(acc[pl.ds(i, _L)], 1.0)

    @pl.loop(0, _CPW)
    def _(k):
        ci = g * _CPW + k
        _zero_acc()

        @pl.loop(0, _NCHUNK)
        def _(ch):
            pltpu.sync_copy(idx_hbm.at[pl.ds(b * _N + ch * _CHUNK, _CHUNK)], idxb)
            pltpu.sync_copy(
                feat_hbm.at[pl.ds((b * _C + ci) * _N + ch * _CHUNK, _CHUNK)], featb
            )

            @pl.loop(0, _CHUNK, step=_L)
            def _(i):
                vidx = idxb[pl.ds(i, _L)]
                v = featb[pl.ds(i, _L)]
                plsc.addupdate_scatter(acc, [vidx], v)

        @pl.loop(0, _R3, step=_L)
        def _(i):
            acc[pl.ds(i, _L)] = acc[pl.ds(i, _L)] * invc[pl.ds(i, _L)]

        pltpu.sync_copy(acc, out_hbm.at[pl.ds((b * _C + ci) * _R3, _R3)])


_sc_params = pltpu.CompilerParams()
if "needs_layout_passes" in pltpu.CompilerParams.__dataclass_fields__:
    _sc_params = dataclasses.replace(_sc_params, needs_layout_passes=False)

_sc_seg = pl.kernel(
    _sc_body,
    out_type=jax.ShapeDtypeStruct((_B * _C * _R3,), jnp.float32),
    mesh=plsc.VectorSubcoreMesh(core_axis_name="c", subcore_axis_name="s"),
    compiler_params=_sc_params,
    scratch_types=[
        pltpu.VMEM((_R3,), jnp.float32),
        pltpu.VMEM((_R3,), jnp.float32),
        pltpu.VMEM((_CHUNK,), jnp.int32),
        pltpu.VMEM((_CHUNK,), jnp.float32),
    ],
)


def kernel(features, coords):
    ncoords, idx = _prep(coords)
    segavg = _sc_seg(features.reshape(_B * _C * _N), idx.reshape(_B * _N))
    return segavg.reshape(_B, _C, _R, _R, _R), ncoords


# speedup vs baseline: 2.3573x; 1.9275x over previous
import dataclasses
import functools

import jax
import jax.numpy as jnp
from jax import lax
from jax.experimental import pallas as pl
from jax.experimental.pallas import tpu as pltpu
from jax.experimental.pallas import tpu_sc as plsc

_R = 32
_R3 = _R * _R * _R
_B = 4
_C = 64
_N = 100000
_CHUNK = 10000
_NCHUNK = _N // _CHUNK
_NC = 2
_NS = 16
_NW = _NC * _NS
_WPB = _NW // _B
_CPW = _C // _WPB
_L = 16
_U_CH = 5 * _L
_U_R3 = 4 * _L
assert _CHUNK % _U_CH == 0 and _R3 % _U_R3 == 0


def _prep_body(c_ref, nc_ref, idx_ref):
    c = c_ref[0]
    mean = jnp.mean(c, axis=1, keepdims=True)
    nrm = c - mean
    sumsq = jnp.sum(nrm * nrm, axis=0, keepdims=True)
    rmax = jnp.sqrt(jnp.max(sumsq))
    ncoords = nrm / (rmax * 2.0) + 0.5
    scaled = jnp.clip(ncoords * _R, 0.0, _R - 1.0)
    nc_ref[0] = scaled
    vox = jnp.round(scaled).astype(jnp.int32)
    idx = vox[0:1] * (_R * _R) + vox[1:2] * _R + vox[2:3]
    idx_ref[0] = idx


def _prep(coords):
    return pl.pallas_call(
        _prep_body,
        grid=(_B,),
        in_specs=[pl.BlockSpec((1, 3, _N), lambda b: (b, 0, 0))],
        out_specs=[
            pl.BlockSpec((1, 3, _N), lambda b: (b, 0, 0)),
            pl.BlockSpec((1, 1, _N), lambda b: (b, 0, 0)),
        ],
        out_shape=[
            jax.ShapeDtypeStruct((_B, 3, _N), jnp.float32),
            jax.ShapeDtypeStruct((_B, 1, _N), jnp.int32),
        ],
    )(coords)


def _sc_body(
    feat_hbm, idx_hbm, out_hbm, acc, invc, idxb0, idxb1, featb0, featb1,
    isem0, isem1, fsem0, fsem1,
):
    cid = lax.axis_index("c")
    sid = lax.axis_index("s")
    wid = sid * _NC + cid
    b = wid // _WPB
    g = wid % _WPB

    ones = jnp.ones((_L,), jnp.float32)
    idxb = (idxb0, idxb1)
    featb = (featb0, featb1)
    isem = (isem0, isem1)
    fsem = (fsem0, fsem1)

    def _idx_copy(ch, s):
        return pltpu.make_async_copy(
            idx_hbm.at[pl.ds(b * _N + ch * _CHUNK, _CHUNK)],
            idxb[s],
            isem[s],
        )

    def _feat_copy(ci, ch, s):
        return pltpu.make_async_copy(
            feat_hbm.at[pl.ds((b * _C + ci) * _N + ch * _CHUNK, _CHUNK)],
            featb[s],
            fsem[s],
        )

    def _zero(ref):
        @pl.loop(0, _R3, step=_U_R3)
        def _(i):
            for u in range(0, _U_R3, _L):
                ref[pl.ds(i + u, _L)] = jnp.zeros((_L,), jnp.float32)

    def _scatter_chunk(s, with_counts):
        @pl.loop(0, _CHUNK, step=_U_CH)
        def _(i):
            for u in range(0, _U_CH, _L):
                vidx = idxb[s][pl.ds(i + u, _L)]
                v = featb[s][pl.ds(i + u, _L)]
                plsc.addupdate_scatter(acc, [vidx], v)
                if with_counts:
                    plsc.addupdate_scatter(invc, [vidx], ones)

    def _channel_pass(ci, with_counts):
        _zero(acc)
        _idx_copy(0, 0).start()
        _feat_copy(ci, 0, 0).start()
        _idx_copy(1, 1).start()
        _feat_copy(ci, 1, 1).start()

        @pl.loop(0, _NCHUNK - 2, step=2)
        def _(ch):
            _idx_copy(ch, 0).wait()
            _feat_copy(ci, ch, 0).wait()
            _scatter_chunk(0, with_counts)
            _idx_copy(ch + 2, 0).start()
            _feat_copy(ci, ch + 2, 0).start()

            _idx_copy(ch + 1, 1).wait()
            _feat_copy(ci, ch + 1, 1).wait()
            _scatter_chunk(1, with_counts)
            _idx_copy(ch + 3, 1).start()
            _feat_copy(ci, ch + 3, 1).start()

        _idx_copy(_NCHUNK - 2, 0).wait()
        _feat_copy(ci, _NCHUNK - 2, 0).wait()
        _scatter_chunk(0, with_counts)
        _idx_copy(_NCHUNK - 1, 1).wait()
        _feat_copy(ci, _NCHUNK - 1, 1).wait()
        _scatter_chunk(1, with_counts)

        if with_counts:
            @pl.loop(0, _R3, step=_U_R3)
            def _(i):
                for u in range(0, _U_R3, _L):
                    sl = pl.ds(i + u, _L)
                    invc[sl] = 1.0 / jnp.maximum(invc[sl], 1.0)

        @pl.loop(0, _R3, step=_U_R3)
        def _(i):
            for u in range(0, _U_R3, _L):
                sl = pl.ds(i + u, _L)
                acc[sl] = acc[sl] * invc[sl]

        pltpu.sync_copy(acc, out_hbm.at[pl.ds((b * _C + ci) * _R3, _R3)])

    _zero(invc)
    _channel_pass(g * _CPW, True)

    @pl.loop(1, _CPW)
    def _(k):
        _channel_pass(g * _CPW + k, False)


_sc_params = pltpu.CompilerParams()
if "needs_layout_passes" in pltpu.CompilerParams.__dataclass_fields__:
    _sc_params = dataclasses.replace(_sc_params, needs_layout_passes=False)

_sc_seg = pl.kernel(
    _sc_body,
    out_type=jax.ShapeDtypeStruct((_B * _C * _R3,), jnp.float32),
    mesh=plsc.VectorSubcoreMesh(core_axis_name="c", subcore_axis_name="s"),
    compiler_params=_sc_params,
    scratch_types=[
        pltpu.VMEM((_R3,), jnp.float32),
        pltpu.VMEM((_R3,), jnp.float32),
        pltpu.VMEM((_CHUNK,), jnp.int32),
        pltpu.VMEM((_CHUNK,), jnp.int32),
        pltpu.VMEM((_CHUNK,), jnp.float32),
        pltpu.VMEM((_CHUNK,), jnp.float32),
        pltpu.SemaphoreType.DMA,
        pltpu.SemaphoreType.DMA,
        pltpu.SemaphoreType.DMA,
        pltpu.SemaphoreType.DMA,
    ],
)


def kernel(features, coords):
    ncoords, idx = _prep(coords)
    segavg = _sc_seg(features.reshape(_B * _C * _N), idx.reshape(_B * _N))
    return segavg.reshape(_B, _C, _R, _R, _R), ncoords


# speedup vs baseline: 2.5647x; 1.0880x over previous
import dataclasses
import functools

import jax
import jax.numpy as jnp
from jax import lax
from jax.experimental import pallas as pl
from jax.experimental.pallas import tpu as pltpu
from jax.experimental.pallas import tpu_sc as plsc

_R = 32
_R3 = _R * _R * _R
_B = 4
_C = 64
_N = 100000
_CHUNK = 2000
_NCHUNK = _N // _CHUNK
_NC = 2
_NS = 16
_NW = _NC * _NS
_WPB = _NW // _B
_CPW = _C // _WPB
_NPAIR = _CPW // 2
_L = 16
_U_CH = 5 * _L
_U_R3 = 4 * _L
assert _CHUNK % _U_CH == 0 and _R3 % _U_R3 == 0 and _NCHUNK % 2 == 0


def _prep_body(c_ref, nc_ref, idx_ref):
    c = c_ref[0]
    mean = jnp.mean(c, axis=1, keepdims=True)
    nrm = c - mean
    sumsq = jnp.sum(nrm * nrm, axis=0, keepdims=True)
    rmax = jnp.sqrt(jnp.max(sumsq))
    ncoords = nrm / (rmax * 2.0) + 0.5
    scaled = jnp.clip(ncoords * _R, 0.0, _R - 1.0)
    nc_ref[0] = scaled
    vox = jnp.round(scaled).astype(jnp.int32)
    idx = vox[0:1] * (_R * _R) + vox[1:2] * _R + vox[2:3]
    idx_ref[0] = idx


def _prep(coords):
    return pl.pallas_call(
        _prep_body,
        grid=(_B,),
        in_specs=[pl.BlockSpec((1, 3, _N), lambda b: (b, 0, 0))],
        out_specs=[
            pl.BlockSpec((1, 3, _N), lambda b: (b, 0, 0)),
            pl.BlockSpec((1, 1, _N), lambda b: (b, 0, 0)),
        ],
        out_shape=[
            jax.ShapeDtypeStruct((_B, 3, _N), jnp.float32),
            jax.ShapeDtypeStruct((_B, 1, _N), jnp.int32),
        ],
    )(coords)


def _sc_body(
    feat_hbm, idx_hbm, out_hbm, acc0, acc1, invc,
    idxb0, idxb1, fab0, fab1, fbb0, fbb1,
    isem0, isem1, fasem0, fasem1, fbsem0, fbsem1,
):
    cid = lax.axis_index("c")
    sid = lax.axis_index("s")
    wid = sid * _NC + cid
    b = wid // _WPB
    g = wid % _WPB

    ones = jnp.ones((_L,), jnp.float32)
    idxb = (idxb0, idxb1)
    fab = (fab0, fab1)
    fbb = (fbb0, fbb1)
    isem = (isem0, isem1)
    fasem = (fasem0, fasem1)
    fbsem = (fbsem0, fbsem1)

    def _idx_copy(ch, s):
        return pltpu.make_async_copy(
            idx_hbm.at[pl.ds(b * _N + ch * _CHUNK, _CHUNK)],
            idxb[s],
            isem[s],
        )

    def _feat_copy(ci, ch, s, buf, sem):
        return pltpu.make_async_copy(
            feat_hbm.at[pl.ds((b * _C + ci) * _N + ch * _CHUNK, _CHUNK)],
            buf[s],
            sem[s],
        )

    def _zero(ref):
        @pl.loop(0, _R3, step=_U_R3)
        def _(i):
            for u in range(0, _U_R3, _L):
                ref[pl.ds(i + u, _L)] = jnp.zeros((_L,), jnp.float32)

    def _scatter_chunk(s, with_counts):
        @pl.loop(0, _CHUNK, step=_U_CH)
        def _(i):
            for u in range(0, _U_CH, _L):
                sl = pl.ds(i + u, _L)
                vidx = idxb[s][sl]
                plsc.addupdate_scatter(acc0, [vidx], fab[s][sl])
                plsc.addupdate_scatter(acc1, [vidx], fbb[s][sl])
                if with_counts:
                    plsc.addupdate_scatter(invc, [vidx], ones)

    def _pair_pass(ca, with_counts):
        cb = ca + 1
        _zero(acc0)
        _zero(acc1)
        _idx_copy(0, 0).start()
        _feat_copy(ca, 0, 0, fab, fasem).start()
        _feat_copy(cb, 0, 0, fbb, fbsem).start()
        _idx_copy(1, 1).start()
        _feat_copy(ca, 1, 1, fab, fasem).start()
        _feat_copy(cb, 1, 1, fbb, fbsem).start()

        @pl.loop(0, _NCHUNK - 2, step=2)
        def _(ch):
            _idx_copy(ch, 0).wait()
            _feat_copy(ca, ch, 0, fab, fasem).wait()
            _feat_copy(cb, ch, 0, fbb, fbsem).wait()
            _scatter_chunk(0, with_counts)
            _idx_copy(ch + 2, 0).start()
            _feat_copy(ca, ch + 2, 0, fab, fasem).start()
            _feat_copy(cb, ch + 2, 0, fbb, fbsem).start()

            _idx_copy(ch + 1, 1).wait()
            _feat_copy(ca, ch + 1, 1, fab, fasem).wait()
            _feat_copy(cb, ch + 1, 1, fbb, fbsem).wait()
            _scatter_chunk(1, with_counts)
            _idx_copy(ch + 3, 1).start()
            _feat_copy(ca, ch + 3, 1, fab, fasem).start()
            _feat_copy(cb, ch + 3, 1, fbb, fbsem).start()

        _idx_copy(_NCHUNK - 2, 0).wait()
        _feat_copy(ca, _NCHUNK - 2, 0, fab, fasem).wait()
        _feat_copy(cb, _NCHUNK - 2, 0, fbb, fbsem).wait()
        _scatter_chunk(0, with_counts)
        _idx_copy(_NCHUNK - 1, 1).wait()
        _feat_copy(ca, _NCHUNK - 1, 1, fab, fasem).wait()
        _feat_copy(cb, _NCHUNK - 1, 1, fbb, fbsem).wait()
        _scatter_chunk(1, with_counts)

        if with_counts:
            @pl.loop(0, _R3, step=_U_R3)
            def _(i):
                for u in range(0, _U_R3, _L):
                    sl = pl.ds(i + u, _L)
                    invc[sl] = 1.0 / jnp.maximum(invc[sl], 1.0)

        @pl.loop(0, _R3, step=_U_R3)
        def _(i):
            for u in range(0, _U_R3, _L):
                sl = pl.ds(i + u, _L)
                acc0[sl] = acc0[sl] * invc[sl]
                acc1[sl] = acc1[sl] * invc[sl]

        pltpu.sync_copy(acc0, out_hbm.at[pl.ds((b * _C + ca) * _R3, _R3)])
        pltpu.sync_copy(acc1, out_hbm.at[pl.ds((b * _C + cb) * _R3, _R3)])

    _zero(invc)
    _pair_pass(g * _CPW, True)

    @pl.loop(1, _NPAIR)
    def _(p):
        _pair_pass(g * _CPW + 2 * p, False)


_sc_params = pltpu.CompilerParams()
if "needs_layout_passes" in pltpu.CompilerParams.__dataclass_fields__:
    _sc_params = dataclasses.replace(_sc_params, needs_layout_passes=False)

_sc_seg = pl.kernel(
    _sc_body,
    out_type=jax.ShapeDtypeStruct((_B * _C * _R3,), jnp.float32),
    mesh=plsc.VectorSubcoreMesh(core_axis_name="c", subcore_axis_name="s"),
    compiler_params=_sc_params,
    scratch_types=[
        pltpu.VMEM((_R3,), jnp.float32),
        pltpu.VMEM((_R3,), jnp.float32),
        pltpu.VMEM((_R3,), jnp.float32),
        pltpu.VMEM((_CHUNK,), jnp.int32),
        pltpu.VMEM((_CHUNK,), jnp.int32),
        pltpu.VMEM((_CHUNK,), jnp.float32),
        pltpu.VMEM((_CHUNK,), jnp.float32),
        pltpu.VMEM((_CHUNK,), jnp.float32),
        pltpu.VMEM((_CHUNK,), jnp.float32),
        pltpu.SemaphoreType.DMA,
        pltpu.SemaphoreType.DMA,
        pltpu.SemaphoreType.DMA,
        pltpu.SemaphoreType.DMA,
        pltpu.SemaphoreType.DMA,
        pltpu.SemaphoreType.DMA,
    ],
)


def kernel(features, coords):
    ncoords, idx = _prep(coords)
    segavg = _sc_seg(features.reshape(_B * _C * _N), idx.reshape(_B * _N))
    return segavg.reshape(_B, _C, _R, _R, _R), ncoords


# speedup vs baseline: 3.1929x; 1.2450x over previous
import dataclasses
import functools

import jax
import jax.numpy as jnp
from jax import lax
from jax.experimental import pallas as pl
from jax.experimental.pallas import tpu as pltpu
from jax.experimental.pallas import tpu_sc as plsc

_R = 32
_R3 = _R * _R * _R
_B = 4
_C = 64
_N = 100000
_CHUNK = 2000
_NCHUNK = _N // _CHUNK
_NC = 2
_NS = 16
_NW = _NC * _NS
_WPB = _NW // _B
_CPW = _C // _WPB
_NPAIR = _CPW // 2
_L = 16
_U_CH = 5 * _L
_U_R3 = 4 * _L
assert _CHUNK % _U_CH == 0 and _R3 % _U_R3 == 0 and _NCHUNK % 2 == 0


def _prep_body(c_ref, nc_ref, idx_ref):
    c = c_ref[0]
    mean = jnp.mean(c, axis=1, keepdims=True)
    nrm = c - mean
    sumsq = jnp.sum(nrm * nrm, axis=0, keepdims=True)
    rmax = jnp.sqrt(jnp.max(sumsq))
    ncoords = nrm / (rmax * 2.0) + 0.5
    scaled = jnp.clip(ncoords * _R, 0.0, _R - 1.0)
    nc_ref[0] = scaled
    vox = jnp.round(scaled).astype(jnp.int32)
    idx = vox[0:1] * (_R * _R) + vox[1:2] * _R + vox[2:3]
    idx_ref[0] = idx


def _prep(coords):
    return pl.pallas_call(
        _prep_body,
        grid=(_B,),
        in_specs=[pl.BlockSpec((1, 3, _N), lambda b: (b, 0, 0))],
        out_specs=[
            pl.BlockSpec((1, 3, _N), lambda b: (b, 0, 0)),
            pl.BlockSpec((1, 1, _N), lambda b: (b, 0, 0)),
        ],
        out_shape=[
            jax.ShapeDtypeStruct((_B, 3, _N), jnp.float32),
            jax.ShapeDtypeStruct((_B, 1, _N), jnp.int32),
        ],
    )(coords)


def _sc_body(
    feat_hbm, idx_hbm, out_hbm, acc0, acc1, invc,
    idxb0, idxb1, fab0, fab1, fbb0, fbb1,
    isem0, isem1, fasem0, fasem1, fbsem0, fbsem1,
):
    cid = lax.axis_index("c")
    sid = lax.axis_index("s")
    wid = sid * _NC + cid
    b = wid // _WPB
    g = wid % _WPB

    ones = jnp.ones((_L,), jnp.float32)
    idxb = (idxb0, idxb1)
    fab = (fab0, fab1)
    fbb = (fbb0, fbb1)
    isem = (isem0, isem1)
    fasem = (fasem0, fasem1)
    fbsem = (fbsem0, fbsem1)

    def _idx_copy(ch, s):
        return pltpu.make_async_copy(
            idx_hbm.at[pl.ds(b * _N + ch * _CHUNK, _CHUNK)],
            idxb[s],
            isem[s],
        )

    def _feat_copy(ci, ch, s, buf, sem):
        return pltpu.make_async_copy(
            feat_hbm.at[pl.ds((b * _C + ci) * _N + ch * _CHUNK, _CHUNK)],
            buf[s],
            sem[s],
        )

    def _zero(ref):
        @plsc.parallel_loop(0, _R3, _L, unroll=4)
        def _(i):
            ref[pl.ds(i, _L)] = jnp.zeros((_L,), jnp.float32)

    def _scatter_chunk(s, with_counts):
        @plsc.parallel_loop(0, _CHUNK, _L, unroll=5)
        def _(i):
            sl = pl.ds(i, _L)
            vidx = idxb[s][sl]
            plsc.addupdate_scatter(acc0, [vidx], fab[s][sl])
            plsc.addupdate_scatter(acc1, [vidx], fbb[s][sl])
            if with_counts:
                plsc.addupdate_scatter(invc, [vidx], ones)

    def _pair_pass(ca, with_counts):
        cb = ca + 1
        _zero(acc0)
        _zero(acc1)
        _idx_copy(0, 0).start()
        _feat_copy(ca, 0, 0, fab, fasem).start()
        _feat_copy(cb, 0, 0, fbb, fbsem).start()
        _idx_copy(1, 1).start()
        _feat_copy(ca, 1, 1, fab, fasem).start()
        _feat_copy(cb, 1, 1, fbb, fbsem).start()

        @pl.loop(0, _NCHUNK - 2, step=2)
        def _(ch):
            _idx_copy(ch, 0).wait()
            _feat_copy(ca, ch, 0, fab, fasem).wait()
            _feat_copy(cb, ch, 0, fbb, fbsem).wait()
            _scatter_chunk(0, with_counts)
            _idx_copy(ch + 2, 0).start()
            _feat_copy(ca, ch + 2, 0, fab, fasem).start()
            _feat_copy(cb, ch + 2, 0, fbb, fbsem).start()

            _idx_copy(ch + 1, 1).wait()
            _feat_copy(ca, ch + 1, 1, fab, fasem).wait()
            _feat_copy(cb, ch + 1, 1, fbb, fbsem).wait()
            _scatter_chunk(1, with_counts)
            _idx_copy(ch + 3, 1).start()
            _feat_copy(ca, ch + 3, 1, fab, fasem).start()
            _feat_copy(cb, ch + 3, 1, fbb, fbsem).start()

        _idx_copy(_NCHUNK - 2, 0).wait()
        _feat_copy(ca, _NCHUNK - 2, 0, fab, fasem).wait()
        _feat_copy(cb, _NCHUNK - 2, 0, fbb, fbsem).wait()
        _scatter_chunk(0, with_counts)
        _idx_copy(_NCHUNK - 1, 1).wait()
        _feat_copy(ca, _NCHUNK - 1, 1, fab, fasem).wait()
        _feat_copy(cb, _NCHUNK - 1, 1, fbb, fbsem).wait()
        _scatter_chunk(1, with_counts)

        if with_counts:
            @plsc.parallel_loop(0, _R3, _L, unroll=4)
            def _(i):
                sl = pl.ds(i, _L)
                invc[sl] = 1.0 / jnp.maximum(invc[sl], 1.0)

        @plsc.parallel_loop(0, _R3, _L, unroll=4)
        def _(i):
            sl = pl.ds(i, _L)
            acc0[sl] = acc0[sl] * invc[sl]
            acc1[sl] = acc1[sl] * invc[sl]

        pltpu.sync_copy(acc0, out_hbm.at[pl.ds((b * _C + ca) * _R3, _R3)])
        pltpu.sync_copy(acc1, out_hbm.at[pl.ds((b * _C + cb) * _R3, _R3)])

    _zero(invc)
    _pair_pass(g * _CPW, True)

    @pl.loop(1, _NPAIR)
    def _(p):
        _pair_pass(g * _CPW + 2 * p, False)


_sc_params = pltpu.CompilerParams()
if "needs_layout_passes" in pltpu.CompilerParams.__dataclass_fields__:
    _sc_params = dataclasses.replace(_sc_params, needs_layout_passes=False)

_sc_seg = pl.kernel(
    _sc_body,
    out_type=jax.ShapeDtypeStruct((_B * _C * _R3,), jnp.float32),
    mesh=plsc.VectorSubcoreMesh(core_axis_name="c", subcore_axis_name="s"),
    compiler_params=_sc_params,
    scratch_types=[
        pltpu.VMEM((_R3,), jnp.float32),
        pltpu.VMEM((_R3,), jnp.float32),
        pltpu.VMEM((_R3,), jnp.float32),
        pltpu.VMEM((_CHUNK,), jnp.int32),
        pltpu.VMEM((_CHUNK,), jnp.int32),
        pltpu.VMEM((_CHUNK,), jnp.float32),
        pltpu.VMEM((_CHUNK,), jnp.float32),
        pltpu.VMEM((_CHUNK,), jnp.float32),
        pltpu.VMEM((_CHUNK,), jnp.float32),
        pltpu.SemaphoreType.DMA,
        pltpu.SemaphoreType.DMA,
        pltpu.SemaphoreType.DMA,
        pltpu.SemaphoreType.DMA,
        pltpu.SemaphoreType.DMA,
        pltpu.SemaphoreType.DMA,
    ],
)


def kernel(features, coords):
    ncoords, idx = _prep(coords)
    segavg = _sc_seg(features.reshape(_B * _C * _N), idx.reshape(_B * _N))
    return segavg.reshape(_B, _C, _R, _R, _R), ncoords


# speedup vs baseline: 3.2504x; 1.0180x over previous
import dataclasses
import functools

import jax
import jax.numpy as jnp
from jax import lax
from jax.experimental import pallas as pl
from jax.experimental.pallas import tpu as pltpu
from jax.experimental.pallas import tpu_sc as plsc

_R = 32
_R3 = _R * _R * _R
_B = 4
_C = 64
_N = 100000
_CHUNK = 2000
_NCHUNK = _N // _CHUNK
_NC = 2
_NS = 16
_NW = _NC * _NS
_WPB = _NW // _B
_CPW = _C // _WPB
_NPAIR = _CPW // 2
_L = 16
_U_CH = 5 * _L
_U_R3 = 4 * _L
assert _CHUNK % _U_CH == 0 and _R3 % _U_R3 == 0 and _NCHUNK % 2 == 0


def _prep_body(c_ref, nc_ref, idx_ref):
    c = c_ref[0]
    mean = jnp.mean(c, axis=1, keepdims=True)
    nrm = c - mean
    sumsq = jnp.sum(nrm * nrm, axis=0, keepdims=True)
    rmax = jnp.sqrt(jnp.max(sumsq))
    ncoords = nrm / (rmax * 2.0) + 0.5
    scaled = jnp.clip(ncoords * _R, 0.0, _R - 1.0)
    nc_ref[0] = scaled
    vox = jnp.round(scaled).astype(jnp.int32)
    idx = vox[0:1] * (_R * _R) + vox[1:2] * _R + vox[2:3]
    idx_ref[0] = idx


def _prep(coords):
    return pl.pallas_call(
        _prep_body,
        grid=(_B,),
        in_specs=[pl.BlockSpec((1, 3, _N), lambda b: (b, 0, 0))],
        out_specs=[
            pl.BlockSpec((1, 3, _N), lambda b: (b, 0, 0)),
            pl.BlockSpec((1, 1, _N), lambda b: (b, 0, 0)),
        ],
        out_shape=[
            jax.ShapeDtypeStruct((_B, 3, _N), jnp.float32),
            jax.ShapeDtypeStruct((_B, 1, _N), jnp.int32),
        ],
    )(coords)


def _sc_body(
    feat_hbm, idx_hbm, out_hbm, acc0, acc1, invc,
    idxb0, idxb1, fab0, fab1, fbb0, fbb1,
    isem0, isem1, fasem0, fasem1, fbsem0, fbsem1,
):
    cid = lax.axis_index("c")
    sid = lax.axis_index("s")
    wid = sid * _NC + cid
    b = wid // _WPB
    g = wid % _WPB

    ones = jnp.ones((_L,), jnp.float32)
    idxb = (idxb0, idxb1)
    fab = (fab0, fab1)
    fbb = (fbb0, fbb1)
    isem = (isem0, isem1)
    fasem = (fasem0, fasem1)
    fbsem = (fbsem0, fbsem1)

    def _idx_copy(ch, s):
        return pltpu.make_async_copy(
            idx_hbm.at[pl.ds(b * _N + ch * _CHUNK, _CHUNK)],
            idxb[s],
            isem[s],
        )

    def _feat_copy(ci, ch, s, buf, sem):
        return pltpu.make_async_copy(
            feat_hbm.at[pl.ds((b * _C + ci) * _N + ch * _CHUNK, _CHUNK)],
            buf[s],
            sem[s],
        )

    def _zero(ref):
        @plsc.parallel_loop(0, _R3, _L, unroll=8)
        def _(i):
            ref[pl.ds(i, _L)] = jnp.zeros((_L,), jnp.float32)

    def _scatter_chunk(s, with_counts):
        @plsc.parallel_loop(0, _CHUNK, _L, unroll=25)
        def _(i):
            sl = pl.ds(i, _L)
            vidx = idxb[s][sl]
            plsc.addupdate_scatter(acc0, [vidx], fab[s][sl])
            plsc.addupdate_scatter(acc1, [vidx], fbb[s][sl])
            if with_counts:
                plsc.addupdate_scatter(invc, [vidx], ones)

    def _pair_pass(ca, with_counts):
        cb = ca + 1
        _zero(acc0)
        _zero(acc1)
        _idx_copy(0, 0).start()
        _feat_copy(ca, 0, 0, fab, fasem).start()
        _feat_copy(cb, 0, 0, fbb, fbsem).start()
        _idx_copy(1, 1).start()
        _feat_copy(ca, 1, 1, fab, fasem).start()
        _feat_copy(cb, 1, 1, fbb, fbsem).start()

        @pl.loop(0, _NCHUNK - 2, step=2)
        def _(ch):
            _idx_copy(ch, 0).wait()
            _feat_copy(ca, ch, 0, fab, fasem).wait()
            _feat_copy(cb, ch, 0, fbb, fbsem).wait()
            _scatter_chunk(0, with_counts)
            _idx_copy(ch + 2, 0).start()
            _feat_copy(ca, ch + 2, 0, fab, fasem).start()
            _feat_copy(cb, ch + 2, 0, fbb, fbsem).start()

            _idx_copy(ch + 1, 1).wait()
            _feat_copy(ca, ch + 1, 1, fab, fasem).wait()
            _feat_copy(cb, ch + 1, 1, fbb, fbsem).wait()
            _scatter_chunk(1, with_counts)
            _idx_copy(ch + 3, 1).start()
            _feat_copy(ca, ch + 3, 1, fab, fasem).start()
            _feat_copy(cb, ch + 3, 1, fbb, fbsem).start()

        _idx_copy(_NCHUNK - 2, 0).wait()
        _feat_copy(ca, _NCHUNK - 2, 0, fab, fasem).wait()
        _feat_copy(cb, _NCHUNK - 2, 0, fbb, fbsem).wait()
        _scatter_chunk(0, with_counts)
        _idx_copy(_NCHUNK - 1, 1).wait()
        _feat_copy(ca, _NCHUNK - 1, 1, fab, fasem).wait()
        _feat_copy(cb, _NCHUNK - 1, 1, fbb, fbsem).wait()
        _scatter_chunk(1, with_counts)

        if with_counts:
            @plsc.parallel_loop(0, _R3, _L, unroll=8)
            def _(i):
                sl = pl.ds(i, _L)
                invc[sl] = 1.0 / jnp.maximum(invc[sl], 1.0)

        @plsc.parallel_loop(0, _R3, _L, unroll=8)
        def _(i):
            sl = pl.ds(i, _L)
            acc0[sl] = acc0[sl] * invc[sl]
            acc1[sl] = acc1[sl] * invc[sl]

        pltpu.sync_copy(acc0, out_hbm.at[pl.ds((b * _C + ca) * _R3, _R3)])
        pltpu.sync_copy(acc1, out_hbm.at[pl.ds((b * _C + cb) * _R3, _R3)])

    _zero(invc)
    _pair_pass(g * _CPW, True)

    @pl.loop(1, _NPAIR)
    def _(p):
        _pair_pass(g * _CPW + 2 * p, False)


_sc_params = pltpu.CompilerParams()
if "needs_layout_passes" in pltpu.CompilerParams.__dataclass_fields__:
    _sc_params = dataclasses.replace(_sc_params, needs_layout_passes=False)

_sc_seg = pl.kernel(
    _sc_body,
    out_type=jax.ShapeDtypeStruct((_B * _C * _R3,), jnp.float32),
    mesh=plsc.VectorSubcoreMesh(core_axis_name="c", subcore_axis_name="s"),
    compiler_params=_sc_params,
    scratch_types=[
        pltpu.VMEM((_R3,), jnp.float32),
        pltpu.VMEM((_R3,), jnp.float32),
        pltpu.VMEM((_R3,), jnp.float32),
        pltpu.VMEM((_CHUNK,), jnp.int32),
        pltpu.VMEM((_CHUNK,), jnp.int32),
        pltpu.VMEM((_CHUNK,), jnp.float32),
        pltpu.VMEM((_CHUNK,), jnp.float32),
        pltpu.VMEM((_CHUNK,), jnp.float32),
        pltpu.VMEM((_CHUNK,), jnp.float32),
        pltpu.SemaphoreType.DMA,
        pltpu.SemaphoreType.DMA,
        pltpu.SemaphoreType.DMA,
        pltpu.SemaphoreType.DMA,
        pltpu.SemaphoreType.DMA,
        pltpu.SemaphoreType.DMA,
    ],
)


def kernel(features, coords):
    ncoords, idx = _prep(coords)
    segavg = _sc_seg(features.reshape(_B * _C * _N), idx.reshape(_B * _N))
    return segavg.reshape(_B, _C, _R, _R, _R), ncoords
